# baseline (speedup 1.0000x reference)
import functools

import jax
import jax.numpy as jnp
from jax import lax
from jax.experimental import pallas as pl
from jax.experimental.pallas import tpu as pltpu
from jax.experimental.pallas import tpu_sc as plsc

N = 10000
E = 320000
F = 128
NUM_LAYERS = 3
BN_EPS = 1e-5
BN_INV = 1.0 / (1.0 + BN_EPS) ** 0.5

NC = 2
NS = 16
NW = NC * NS
CHUNK = 128
GS = 16
NG = 5
NCHUNK = NG * GS
EPW = NCHUNK * CHUNK
EPAD = NW * EPW
AGG_ROWS = 10240
RPT = AGG_ROWS // NS
ZCOPIES = RPT // CHUNK


def _sc_agg_body(h_hbm, src_hbm, dst_hbm, out_hbm, src_v, dst_v, rows_v,
                 agg_sh, sem0, sem1, isem0, isem1):
    c = lax.axis_index("c")
    s = lax.axis_index("s")
    wid = c * NS + s
    sems = (sem0, sem1)
    pltpu.sync_copy(src_hbm.at[wid], src_v)
    pltpu.sync_copy(dst_hbm.at[wid], dst_v)

    def zbody(i, _):
        rows_v[0, i // 8, pl.ds((i % 8) * 16, 16)] = jnp.zeros((16,),
                                                               jnp.float32)
        return 0
    lax.fori_loop(0, CHUNK * 8, zbody, 0)
    base = s * RPT
    for k in range(ZCOPIES):
        pltpu.sync_copy(rows_v.at[0], agg_sh.at[pl.ds(base + k * CHUNK,
                                                      CHUNK)])
    plsc.subcore_barrier()

    def body(j, _):
        pltpu.async_copy(h_hbm.at[src_v.at[j]], rows_v.at[0], sems[0]).wait()
        pltpu.sync_copy(rows_v.at[0], agg_sh.at[dst_v.at[j]], add=True)
        return 0
    lax.fori_loop(0, NCHUNK, body, 0)
    plsc.subcore_barrier()

    pltpu.sync_copy(agg_sh.at[pl.ds(base, RPT)],
                    out_hbm.at[c, pl.ds(base, RPT)])


@functools.cache
def _sc_agg():
    return pl.kernel(
        _sc_agg_body,
        out_type=jax.ShapeDtypeStruct((NC, AGG_ROWS, F), jnp.float32),
        mesh=plsc.VectorSubcoreMesh(core_axis_name="c", subcore_axis_name="s",
                                    num_cores=NC, num_subcores=NS),
        scratch_types=[
            pltpu.VMEM((NCHUNK, CHUNK), jnp.int32),
            pltpu.VMEM((NCHUNK, CHUNK), jnp.int32),
            pltpu.VMEM((1, CHUNK, F), jnp.float32),
            pltpu.VMEM_SHARED((AGG_ROWS, F), jnp.float32),
            pltpu.SemaphoreType.DMA,
            pltpu.SemaphoreType.DMA,
            pltpu.SemaphoreType.DMA,
            pltpu.SemaphoreType.DMA,
        ],
    )


def _mlp_body(relu_last, h_ref, a0_ref, a1_ref, wa_ref, ba_ref, ga_ref,
              bea_ref, wb_ref, bb_ref, gb_ref, beb_ref, go_ref, beo_ref,
              out_ref):
    m = h_ref[...] + a0_ref[...] + a1_ref[...]
    t = jnp.dot(m, wa_ref[...], preferred_element_type=jnp.float32)
    t = (t + ba_ref[...]) * (ga_ref[...] * BN_INV) + bea_ref[...]
    t = jnp.maximum(t, 0.0)
    t = jnp.dot(t, wb_ref[...], preferred_element_type=jnp.float32)
    t = (t + bb_ref[...]) * (gb_ref[...] * BN_INV) + beb_ref[...]
    if relu_last:
        t = jnp.maximum(t, 0.0)
    t = t * (go_ref[...] * BN_INV) + beo_ref[...]
    if relu_last:
        t = jnp.maximum(t, 0.0)
    out_ref[...] = t


BLK = 1000


def _mlp(h, a0, a1, p, l, last):
    vec = lambda v: v.reshape(1, F)
    args = (h, a0, a1,
            p['w%da' % l], vec(p['b%da' % l]), vec(p['g%da' % l]),
            vec(p['be%da' % l]),
            p['w%db' % l], vec(p['b%db' % l]), vec(p['g%db' % l]),
            vec(p['be%db' % l]),
            vec(p['g%do' % l]), vec(p['be%do' % l]))
    row_spec = pl.BlockSpec((BLK, F), lambda i: (i, 0))
    full = lambda a: pl.BlockSpec(a.shape, lambda i: (0, 0))
    return pl.pallas_call(
        functools.partial(_mlp_body, not last),
        grid=(N // BLK,),
        in_specs=[row_spec, row_spec, row_spec] + [full(a) for a in args[3:]],
        out_specs=row_spec,
        out_shape=jax.ShapeDtypeStruct((N, F), jnp.float32),
    )(*args)


def kernel(x, edge_index, params):
    src = edge_index[0].astype(jnp.int32)
    dst = edge_index[1].astype(jnp.int32)
    pad = EPAD - E
    src = jnp.concatenate([src, jnp.zeros((pad,), jnp.int32)])
    dst = jnp.concatenate([dst, jnp.full((pad,), N, jnp.int32)])
    src = src.reshape(NW, NCHUNK, CHUNK)
    dst = dst.reshape(NW, NCHUNK, CHUNK)
    h = x
    for l in range(NUM_LAYERS):
        aggs = _sc_agg()(h, src, dst)
        h = _mlp(h, aggs[0], aggs[1], params, l, last=(l == NUM_LAYERS - 1))
    return h


# speedup vs baseline: 1.1827x; 1.1827x over previous
import functools

import jax
import jax.numpy as jnp
from jax import lax
from jax.experimental import pallas as pl
from jax.experimental.pallas import tpu as pltpu
from jax.experimental.pallas import tpu_sc as plsc

N = 10000
E = 320000
F = 128
NUM_LAYERS = 3
BN_EPS = 1e-5
BN_INV = 1.0 / (1.0 + BN_EPS) ** 0.5

NC = 2
NS = 16
CHUNK = 128
GS = 16
NG0 = 8
NG1 = 2
NGMAX = max(NG0, NG1)
Q0, Q1 = NG0 * GS, NG1 * GS
CAP0 = NS * Q0 * CHUNK
CAP1 = NS * Q1 * CHUNK
EPAD = CAP0 + CAP1
AGG_ROWS = 10240
RPT = AGG_ROWS // NS
ZCOPIES = RPT // CHUNK


def _sc_agg_body(h_hbm, src_hbm, dst_hbm, out_hbm, src_g, dst_g, rows_v,
                 agg_sh, sem0, sem1, isem0, isem1):
    c = lax.axis_index("c")
    s = lax.axis_index("s")
    sems = (sem0, sem1)
    ng = jnp.where(c == 0, NG0, NG1)

    pltpu.sync_copy(src_hbm.at[c, s, pl.ds(0, GS)], src_g.at[0])
    pltpu.sync_copy(dst_hbm.at[c, s, pl.ds(0, GS)], dst_g.at[0])

    def zbody(i, _):
        rows_v[0, i // 8, pl.ds((i % 8) * 16, 16)] = jnp.zeros((16,),
                                                               jnp.float32)
        return 0
    lax.fori_loop(0, CHUNK * 8, zbody, 0)
    base = s * RPT
    for k in range(ZCOPIES):
        pltpu.sync_copy(rows_v.at[0], agg_sh.at[pl.ds(base + k * CHUNK,
                                                      CHUNK)])
    pltpu.async_copy(h_hbm.at[src_g.at[0, 0]], rows_v.at[0], sems[0])
    plsc.subcore_barrier()

    for g in range(NGMAX):
        gb = g % 2

        if g + 1 < NGMAX:
            @pl.when(g + 1 < ng)
            def _load_next(gb=gb, g=g):
                pltpu.async_copy(src_hbm.at[c, s, pl.ds((g + 1) * GS, GS)],
                                 src_g.at[1 - gb], isem0)
                pltpu.async_copy(dst_hbm.at[c, s, pl.ds((g + 1) * GS, GS)],
                                 dst_g.at[1 - gb], isem1)

        @pl.when(g < ng)
        def _run_group(gb=gb):
            def pair_body(i, _):
                for b in range(2):
                    k = 2 * i + b

                    @pl.when(k + 1 < GS)
                    def _start():
                        pltpu.async_copy(h_hbm.at[src_g.at[gb, k + 1]],
                                         rows_v.at[1 - b], sems[1 - b])

                    pltpu.make_async_copy(h_hbm.at[src_g.at[gb, k]],
                                          rows_v.at[b], sems[b]).wait()
                    pltpu.sync_copy(rows_v.at[b], agg_sh.at[dst_g.at[gb, k]],
                                    add=True)
                return 0
            lax.fori_loop(0, GS // 2, pair_body, 0)

        if g + 1 < NGMAX:
            @pl.when(g + 1 < ng)
            def _next_group_prime(gb=gb, g=g):
                pltpu.make_async_copy(
                    src_hbm.at[c, s, pl.ds((g + 1) * GS, GS)],
                    src_g.at[1 - gb], isem0).wait()
                pltpu.make_async_copy(
                    dst_hbm.at[c, s, pl.ds((g + 1) * GS, GS)],
                    dst_g.at[1 - gb], isem1).wait()
                pltpu.async_copy(h_hbm.at[src_g.at[1 - gb, 0]], rows_v.at[0],
                                 sems[0])
    plsc.subcore_barrier()

    pltpu.sync_copy(agg_sh.at[pl.ds(base, RPT)],
                    out_hbm.at[c, pl.ds(base, RPT)])


@functools.cache
def _sc_agg():
    return pl.kernel(
        _sc_agg_body,
        out_type=jax.ShapeDtypeStruct((NC, AGG_ROWS, F), jnp.float32),
        mesh=plsc.VectorSubcoreMesh(core_axis_name="c", subcore_axis_name="s",
                                    num_cores=NC, num_subcores=NS),
        scratch_types=[
            pltpu.VMEM((2, GS, CHUNK), jnp.int32),
            pltpu.VMEM((2, GS, CHUNK), jnp.int32),
            pltpu.VMEM((2, CHUNK, F), jnp.float32),
            pltpu.VMEM_SHARED((AGG_ROWS, F), jnp.float32),
            pltpu.SemaphoreType.DMA,
            pltpu.SemaphoreType.DMA,
            pltpu.SemaphoreType.DMA,
            pltpu.SemaphoreType.DMA,
        ],
    )


def _mlp_body(relu_last, h_ref, a0_ref, a1_ref, wa_ref, ba_ref, ga_ref,
              bea_ref, wb_ref, bb_ref, gb_ref, beb_ref, go_ref, beo_ref,
              out_ref):
    m = h_ref[...] + a0_ref[...] + a1_ref[...]
    t = jnp.dot(m, wa_ref[...], preferred_element_type=jnp.float32)
    t = (t + ba_ref[...]) * (ga_ref[...] * BN_INV) + bea_ref[...]
    t = jnp.maximum(t, 0.0)
    t = jnp.dot(t, wb_ref[...], preferred_element_type=jnp.float32)
    t = (t + bb_ref[...]) * (gb_ref[...] * BN_INV) + beb_ref[...]
    if relu_last:
        t = jnp.maximum(t, 0.0)
    t = t * (go_ref[...] * BN_INV) + beo_ref[...]
    if relu_last:
        t = jnp.maximum(t, 0.0)
    out_ref[...] = t


BLK = 1000


def _mlp(h, a0, a1, p, l, last):
    vec = lambda v: v.reshape(1, F)
    args = (h, a0, a1,
            p['w%da' % l], vec(p['b%da' % l]), vec(p['g%da' % l]),
            vec(p['be%da' % l]),
            p['w%db' % l], vec(p['b%db' % l]), vec(p['g%db' % l]),
            vec(p['be%db' % l]),
            vec(p['g%do' % l]), vec(p['be%do' % l]))
    row_spec = pl.BlockSpec((BLK, F), lambda i: (i, 0))
    full = lambda a: pl.BlockSpec(a.shape, lambda i: (0, 0))
    return pl.pallas_call(
        functools.partial(_mlp_body, not last),
        grid=(N // BLK,),
        in_specs=[row_spec, row_spec, row_spec] + [full(a) for a in args[3:]],
        out_specs=row_spec,
        out_shape=jax.ShapeDtypeStruct((N, F), jnp.float32),
    )(*args)


def _layout_edges(idx, fill):
    pad = EPAD - E
    idx = jnp.concatenate([idx, jnp.full((pad,), fill, jnp.int32)])
    c0 = idx[:CAP0].reshape(NS, Q0, CHUNK)
    c1 = idx[CAP0:].reshape(NS, Q1, CHUNK)
    out = jnp.full((NC, NS, NGMAX * GS, CHUNK), fill, jnp.int32)
    out = out.at[0, :, :Q0].set(c0)
    out = out.at[1, :, :Q1].set(c1)
    return out


def kernel(x, edge_index, params):
    src = _layout_edges(edge_index[0].astype(jnp.int32), 0)
    dst = _layout_edges(edge_index[1].astype(jnp.int32), N)
    h = x
    for l in range(NUM_LAYERS):
        aggs = _sc_agg()(h, src, dst)
        h = _mlp(h, aggs[0], aggs[1], params, l, last=(l == NUM_LAYERS - 1))
    return h


# speedup vs baseline: 3.7795x; 3.1956x over previous
import functools

import jax
import jax.numpy as jnp
from jax import lax
from jax.experimental import pallas as pl
from jax.experimental.pallas import tpu as pltpu
from jax.experimental.pallas import tpu_sc as plsc

N = 10000
E = 320000
F = 128
NUM_LAYERS = 3
BN_EPS = 1e-5
BN_INV = 1.0 / (1.0 + BN_EPS) ** 0.5

NC = 2
NS = 16
CHUNK = 128
GS = 16
NG0 = 5
NG1 = 5
NGMAX = max(NG0, NG1)
Q0, Q1 = NG0 * GS, NG1 * GS
CAP0 = NS * Q0 * CHUNK
CAP1 = NS * Q1 * CHUNK
EPAD = CAP0 + CAP1
AGG_ROWS = 10240
RPT = AGG_ROWS // NS
ZCOPIES = RPT // CHUNK


def _sc_agg_body(h_hbm, src_hbm, dst_hbm, out_hbm, src_g, dst_g, rows_v,
                 agg_sh, sem0, sem1, isem0, isem1):
    c = lax.axis_index("c")
    s = lax.axis_index("s")
    sems = (sem0, sem1)
    ng = jnp.where(c == 0, NG0, NG1)

    pltpu.sync_copy(src_hbm.at[c, s, pl.ds(0, GS)], src_g.at[0])
    pltpu.sync_copy(dst_hbm.at[c, s, pl.ds(0, GS)], dst_g.at[0])

    def zbody(i, _):
        rows_v[0, i // 8, pl.ds((i % 8) * 16, 16)] = jnp.zeros((16,),
                                                               jnp.float32)
        return 0
    lax.fori_loop(0, CHUNK * 8, zbody, 0)
    base = s * RPT
    for k in range(ZCOPIES):
        pltpu.sync_copy(rows_v.at[0], agg_sh.at[pl.ds(base + k * CHUNK,
                                                      CHUNK)])
    pltpu.async_copy(h_hbm.at[src_g.at[0, 0]], rows_v.at[0], sems[0])
    plsc.subcore_barrier()

    for g in range(NGMAX):
        gb = g % 2

        if g + 1 < NGMAX:
            @pl.when(g + 1 < ng)
            def _load_next(gb=gb, g=g):
                pltpu.async_copy(src_hbm.at[c, s, pl.ds((g + 1) * GS, GS)],
                                 src_g.at[1 - gb], isem0)
                pltpu.async_copy(dst_hbm.at[c, s, pl.ds((g + 1) * GS, GS)],
                                 dst_g.at[1 - gb], isem1)

        @pl.when(g < ng)
        def _run_group(gb=gb):
            def pair_body(i, _):
                for b in range(2):
                    k = 2 * i + b

                    @pl.when(k + 1 < GS)
                    def _start():
                        pltpu.async_copy(h_hbm.at[src_g.at[gb, k + 1]],
                                         rows_v.at[1 - b], sems[1 - b])

                    pltpu.make_async_copy(h_hbm.at[src_g.at[gb, k]],
                                          rows_v.at[b], sems[b]).wait()
                    pltpu.sync_copy(rows_v.at[b], agg_sh.at[dst_g.at[gb, k]],
                                    add=True)
                return 0
            lax.fori_loop(0, GS // 2, pair_body, 0)

        if g + 1 < NGMAX:
            @pl.when(g + 1 < ng)
            def _next_group_prime(gb=gb, g=g):
                pltpu.make_async_copy(
                    src_hbm.at[c, s, pl.ds((g + 1) * GS, GS)],
                    src_g.at[1 - gb], isem0).wait()
                pltpu.make_async_copy(
                    dst_hbm.at[c, s, pl.ds((g + 1) * GS, GS)],
                    dst_g.at[1 - gb], isem1).wait()
                pltpu.async_copy(h_hbm.at[src_g.at[1 - gb, 0]], rows_v.at[0],
                                 sems[0])
    plsc.subcore_barrier()

    pltpu.sync_copy(agg_sh.at[pl.ds(base, RPT)],
                    out_hbm.at[c, pl.ds(base, RPT)])


@functools.cache
def _sc_agg():
    return pl.kernel(
        _sc_agg_body,
        out_type=jax.ShapeDtypeStruct((NC, AGG_ROWS, F), jnp.float32),
        mesh=plsc.VectorSubcoreMesh(core_axis_name="c", subcore_axis_name="s",
                                    num_cores=NC, num_subcores=NS),
        scratch_types=[
            pltpu.VMEM((2, GS, CHUNK), jnp.int32),
            pltpu.VMEM((2, GS, CHUNK), jnp.int32),
            pltpu.VMEM((2, CHUNK, F), jnp.float32),
            pltpu.VMEM_SHARED((AGG_ROWS, F), jnp.float32),
            pltpu.SemaphoreType.DMA,
            pltpu.SemaphoreType.DMA,
            pltpu.SemaphoreType.DMA,
            pltpu.SemaphoreType.DMA,
        ],
    )


def _mlp_body(relu_last, h_ref, a0_ref, a1_ref, wa_ref, ba_ref, ga_ref,
              bea_ref, wb_ref, bb_ref, gb_ref, beb_ref, go_ref, beo_ref,
              out_ref):
    m = h_ref[...] + a0_ref[...] + a1_ref[...]
    t = jnp.dot(m, wa_ref[...], preferred_element_type=jnp.float32)
    t = (t + ba_ref[...]) * (ga_ref[...] * BN_INV) + bea_ref[...]
    t = jnp.maximum(t, 0.0)
    t = jnp.dot(t, wb_ref[...], preferred_element_type=jnp.float32)
    t = (t + bb_ref[...]) * (gb_ref[...] * BN_INV) + beb_ref[...]
    if relu_last:
        t = jnp.maximum(t, 0.0)
    t = t * (go_ref[...] * BN_INV) + beo_ref[...]
    if relu_last:
        t = jnp.maximum(t, 0.0)
    out_ref[...] = t


BLK = 1000


def _mlp(h, a0, a1, p, l, last):
    vec = lambda v: v.reshape(1, F)
    args = (h, a0, a1,
            p['w%da' % l], vec(p['b%da' % l]), vec(p['g%da' % l]),
            vec(p['be%da' % l]),
            p['w%db' % l], vec(p['b%db' % l]), vec(p['g%db' % l]),
            vec(p['be%db' % l]),
            vec(p['g%do' % l]), vec(p['be%do' % l]))
    row_spec = pl.BlockSpec((BLK, F), lambda i: (i, 0))
    full = lambda a: pl.BlockSpec(a.shape, lambda i: (0, 0))
    return pl.pallas_call(
        functools.partial(_mlp_body, not last),
        grid=(N // BLK,),
        in_specs=[row_spec, row_spec, row_spec] + [full(a) for a in args[3:]],
        out_specs=row_spec,
        out_shape=jax.ShapeDtypeStruct((N, F), jnp.float32),
    )(*args)


def _layout_edges(idx, fill):
    pad = EPAD - E
    filler = fill + jnp.arange(pad, dtype=jnp.int32) % (AGG_ROWS - N)
    idx = jnp.concatenate([idx, filler])
    c0 = idx[:CAP0].reshape(NS, Q0, CHUNK)
    c1 = idx[CAP0:].reshape(NS, Q1, CHUNK)
    unused = fill + (jnp.arange(NC * NS * NGMAX * GS * CHUNK,
                                dtype=jnp.int32) % (AGG_ROWS - N)
                     ).reshape(NC, NS, NGMAX * GS, CHUNK)
    out = unused
    out = out.at[0, :, :Q0].set(c0)
    out = out.at[1, :, :Q1].set(c1)
    return out


def kernel(x, edge_index, params):
    src = _layout_edges(edge_index[0].astype(jnp.int32), 0)
    dst = _layout_edges(edge_index[1].astype(jnp.int32), N)
    h = x
    for l in range(NUM_LAYERS):
        aggs = _sc_agg()(h, src, dst)
        h = _mlp(h, aggs[0], aggs[1], params, l, last=(l == NUM_LAYERS - 1))
    return h


# speedup vs baseline: 3.8734x; 1.0248x over previous
import functools

import jax
import jax.numpy as jnp
from jax import lax
from jax.experimental import pallas as pl
from jax.experimental.pallas import tpu as pltpu
from jax.experimental.pallas import tpu_sc as plsc

N = 10000
E = 320000
F = 128
NUM_LAYERS = 3
BN_EPS = 1e-5
BN_INV = 1.0 / (1.0 + BN_EPS) ** 0.5

NC = 2
NS = 16
CHUNK = 128
GS = 16
NG0 = 5
NG1 = 5
NGMAX = max(NG0, NG1)
Q0, Q1 = NG0 * GS, NG1 * GS
CAP0 = NS * Q0 * CHUNK
CAP1 = NS * Q1 * CHUNK
EPAD = CAP0 + CAP1
AGG_ROWS = 10240
RPT = AGG_ROWS // NS
ZCOPIES = RPT // CHUNK


def _sc_agg_body(h_hbm, src_hbm, dst_hbm, out_hbm, src_g, dst_g, rows_v,
                 agg_sh, sem0, sem1, isem0, isem1, ssem0, ssem1):
    c = lax.axis_index("c")
    s = lax.axis_index("s")
    sems = (sem0, sem1)
    ssems = (ssem0, ssem1)
    ng = jnp.where(c == 0, NG0, NG1)

    pltpu.sync_copy(src_hbm.at[c, s, pl.ds(0, GS)], src_g.at[0])
    pltpu.sync_copy(dst_hbm.at[c, s, pl.ds(0, GS)], dst_g.at[0])

    def zbody(i, _):
        rows_v[0, i // 8, pl.ds((i % 8) * 16, 16)] = jnp.zeros((16,),
                                                               jnp.float32)
        return 0
    lax.fori_loop(0, CHUNK * 8, zbody, 0)
    base = s * RPT
    for k in range(ZCOPIES):
        pltpu.sync_copy(rows_v.at[0], agg_sh.at[pl.ds(base + k * CHUNK,
                                                      CHUNK)])
    pltpu.async_copy(h_hbm.at[src_g.at[0, 0]], rows_v.at[0], sems[0])
    plsc.subcore_barrier()

    for g in range(NGMAX):
        gb = g % 2

        if g + 1 < NGMAX:
            @pl.when(g + 1 < ng)
            def _load_next(gb=gb, g=g):
                pltpu.async_copy(src_hbm.at[c, s, pl.ds((g + 1) * GS, GS)],
                                 src_g.at[1 - gb], isem0)
                pltpu.async_copy(dst_hbm.at[c, s, pl.ds((g + 1) * GS, GS)],
                                 dst_g.at[1 - gb], isem1)

        @pl.when(g < ng)
        def _run_group(gb=gb, g=g):
            first_group = g == 0

            def pair_body(i, _):
                for b in range(2):
                    k = 2 * i + b

                    @pl.when(k + 1 < GS)
                    def _start():
                        def _wait_scatter():
                            pltpu.make_async_copy(
                                rows_v.at[1 - b],
                                agg_sh.at[dst_g.at[gb, k]],
                                ssems[1 - b]).wait()
                        if first_group:
                            pl.when(k >= 1)(_wait_scatter)
                        else:
                            _wait_scatter()
                        pltpu.async_copy(h_hbm.at[src_g.at[gb, k + 1]],
                                         rows_v.at[1 - b], sems[1 - b])

                    pltpu.make_async_copy(h_hbm.at[src_g.at[gb, k]],
                                          rows_v.at[b], sems[b]).wait()
                    pltpu.async_copy(rows_v.at[b], agg_sh.at[dst_g.at[gb, k]],
                                     ssems[b], add=True)
                return 0
            lax.fori_loop(0, GS // 2, pair_body, 0)

        if g + 1 < NGMAX:
            @pl.when(g + 1 < ng)
            def _next_group_prime(gb=gb, g=g):
                pltpu.make_async_copy(
                    src_hbm.at[c, s, pl.ds((g + 1) * GS, GS)],
                    src_g.at[1 - gb], isem0).wait()
                pltpu.make_async_copy(
                    dst_hbm.at[c, s, pl.ds((g + 1) * GS, GS)],
                    dst_g.at[1 - gb], isem1).wait()
                pltpu.make_async_copy(rows_v.at[0], agg_sh.at[dst_g.at[gb, 0]],
                                      ssems[0]).wait()
                pltpu.async_copy(h_hbm.at[src_g.at[1 - gb, 0]], rows_v.at[0],
                                 sems[0])

    @pl.when(ng > 0)
    def _drain():
        pltpu.make_async_copy(rows_v.at[0], agg_sh.at[dst_g.at[0, 0]],
                              ssems[0]).wait()
        pltpu.make_async_copy(rows_v.at[1], agg_sh.at[dst_g.at[0, 0]],
                              ssems[1]).wait()
    plsc.subcore_barrier()

    pltpu.sync_copy(agg_sh.at[pl.ds(base, RPT)],
                    out_hbm.at[c, pl.ds(base, RPT)])


@functools.cache
def _sc_agg():
    return pl.kernel(
        _sc_agg_body,
        out_type=jax.ShapeDtypeStruct((NC, AGG_ROWS, F), jnp.float32),
        mesh=plsc.VectorSubcoreMesh(core_axis_name="c", subcore_axis_name="s",
                                    num_cores=NC, num_subcores=NS),
        scratch_types=[
            pltpu.VMEM((2, GS, CHUNK), jnp.int32),
            pltpu.VMEM((2, GS, CHUNK), jnp.int32),
            pltpu.VMEM((2, CHUNK, F), jnp.float32),
            pltpu.VMEM_SHARED((AGG_ROWS, F), jnp.float32),
            pltpu.SemaphoreType.DMA,
            pltpu.SemaphoreType.DMA,
            pltpu.SemaphoreType.DMA,
            pltpu.SemaphoreType.DMA,
            pltpu.SemaphoreType.DMA,
            pltpu.SemaphoreType.DMA,
        ],
    )


def _mlp_body(relu_last, h_ref, a0_ref, a1_ref, wa_ref, ba_ref, ga_ref,
              bea_ref, wb_ref, bb_ref, gb_ref, beb_ref, go_ref, beo_ref,
              out_ref):
    m = h_ref[...] + a0_ref[...] + a1_ref[...]
    t = jnp.dot(m, wa_ref[...], preferred_element_type=jnp.float32)
    t = (t + ba_ref[...]) * (ga_ref[...] * BN_INV) + bea_ref[...]
    t = jnp.maximum(t, 0.0)
    t = jnp.dot(t, wb_ref[...], preferred_element_type=jnp.float32)
    t = (t + bb_ref[...]) * (gb_ref[...] * BN_INV) + beb_ref[...]
    if relu_last:
        t = jnp.maximum(t, 0.0)
    t = t * (go_ref[...] * BN_INV) + beo_ref[...]
    if relu_last:
        t = jnp.maximum(t, 0.0)
    out_ref[...] = t


BLK = 1000


def _mlp(h, a0, a1, p, l, last):
    vec = lambda v: v.reshape(1, F)
    args = (h, a0, a1,
            p['w%da' % l], vec(p['b%da' % l]), vec(p['g%da' % l]),
            vec(p['be%da' % l]),
            p['w%db' % l], vec(p['b%db' % l]), vec(p['g%db' % l]),
            vec(p['be%db' % l]),
            vec(p['g%do' % l]), vec(p['be%do' % l]))
    row_spec = pl.BlockSpec((BLK, F), lambda i: (i, 0))
    full = lambda a: pl.BlockSpec(a.shape, lambda i: (0, 0))
    return pl.pallas_call(
        functools.partial(_mlp_body, not last),
        grid=(N // BLK,),
        in_specs=[row_spec, row_spec, row_spec] + [full(a) for a in args[3:]],
        out_specs=row_spec,
        out_shape=jax.ShapeDtypeStruct((N, F), jnp.float32),
    )(*args)


def _layout_edges(idx, fill):
    pad = EPAD - E
    filler = fill + jnp.arange(pad, dtype=jnp.int32) % (AGG_ROWS - N)
    idx = jnp.concatenate([idx, filler])
    c0 = idx[:CAP0].reshape(NS, Q0, CHUNK)
    c1 = idx[CAP0:].reshape(NS, Q1, CHUNK)
    unused = fill + (jnp.arange(NC * NS * NGMAX * GS * CHUNK,
                                dtype=jnp.int32) % (AGG_ROWS - N)
                     ).reshape(NC, NS, NGMAX * GS, CHUNK)
    out = unused
    out = out.at[0, :, :Q0].set(c0)
    out = out.at[1, :, :Q1].set(c1)
    return out


def kernel(x, edge_index, params):
    src = _layout_edges(edge_index[0].astype(jnp.int32), 0)
    dst = _layout_edges(edge_index[1].astype(jnp.int32), N)
    h = x
    for l in range(NUM_LAYERS):
        aggs = _sc_agg()(h, src, dst)
        h = _mlp(h, aggs[0], aggs[1], params, l, last=(l == NUM_LAYERS - 1))
    return h


# speedup vs baseline: 3.9677x; 1.0243x over previous
import functools

import jax
import jax.numpy as jnp
from jax import lax
from jax.experimental import pallas as pl
from jax.experimental.pallas import tpu as pltpu
from jax.experimental.pallas import tpu_sc as plsc

N = 10000
E = 320000
F = 128
NUM_LAYERS = 3
BN_EPS = 1e-5
BN_INV = 1.0 / (1.0 + BN_EPS) ** 0.5

NC = 2
NS = 16
CHUNK = 128
GS = 16
NG0 = 5
NG1 = 5
NGMAX = max(NG0, NG1)
Q0, Q1 = NG0 * GS, NG1 * GS
CAP0 = NS * Q0 * CHUNK
CAP1 = NS * Q1 * CHUNK
EPAD = CAP0 + CAP1
AGG_ROWS = 10240
RPT = AGG_ROWS // NS
ZCOPIES = RPT // CHUNK


def _sc_agg_body(h_hbm, src_hbm, dst_hbm, out_hbm, src_g, dst_g, rows_v,
                 agg_sh, sem0, sem1, isem0, isem1, ssem0, ssem1):
    c = lax.axis_index("c")
    s = lax.axis_index("s")
    sems = (sem0, sem1)
    ssems = (ssem0, ssem1)
    ng = jnp.where(c == 0, NG0, NG1)

    pltpu.sync_copy(src_hbm.at[c, s, pl.ds(0, GS)], src_g.at[0])
    pltpu.sync_copy(dst_hbm.at[c, s, pl.ds(0, GS)], dst_g.at[0])

    def zbody(i, _):
        for j in range(8):
            rows_v[0, i, pl.ds(j * 16, 16)] = jnp.zeros((16,), jnp.float32)
        return 0
    lax.fori_loop(0, CHUNK, zbody, 0)
    base = s * RPT
    for k in range(ZCOPIES):
        pltpu.sync_copy(rows_v.at[0], agg_sh.at[pl.ds(base + k * CHUNK,
                                                      CHUNK)])
    pltpu.async_copy(h_hbm.at[src_g.at[0, 0]], rows_v.at[0], sems[0])
    plsc.subcore_barrier()

    for g in range(NGMAX):
        gb = g % 2

        if g + 1 < NGMAX:
            @pl.when(g + 1 < ng)
            def _load_next(gb=gb, g=g):
                pltpu.async_copy(src_hbm.at[c, s, pl.ds((g + 1) * GS, GS)],
                                 src_g.at[1 - gb], isem0)
                pltpu.async_copy(dst_hbm.at[c, s, pl.ds((g + 1) * GS, GS)],
                                 dst_g.at[1 - gb], isem1)

        @pl.when(g < ng)
        def _run_group(gb=gb, g=g):
            first_group = g == 0

            def pair_body(i, _):
                for b in range(2):
                    k = 2 * i + b

                    @pl.when(k + 1 < GS)
                    def _start():
                        def _wait_scatter():
                            pltpu.make_async_copy(
                                rows_v.at[1 - b],
                                agg_sh.at[dst_g.at[gb, k]],
                                ssems[1 - b]).wait()
                        if first_group:
                            pl.when(k >= 1)(_wait_scatter)
                        else:
                            _wait_scatter()
                        pltpu.async_copy(h_hbm.at[src_g.at[gb, k + 1]],
                                         rows_v.at[1 - b], sems[1 - b])

                    pltpu.make_async_copy(h_hbm.at[src_g.at[gb, k]],
                                          rows_v.at[b], sems[b]).wait()
                    pltpu.async_copy(rows_v.at[b], agg_sh.at[dst_g.at[gb, k]],
                                     ssems[b], add=True)
                return 0
            lax.fori_loop(0, GS // 2, pair_body, 0)

        if g + 1 < NGMAX:
            @pl.when(g + 1 < ng)
            def _next_group_prime(gb=gb, g=g):
                pltpu.make_async_copy(
                    src_hbm.at[c, s, pl.ds((g + 1) * GS, GS)],
                    src_g.at[1 - gb], isem0).wait()
                pltpu.make_async_copy(
                    dst_hbm.at[c, s, pl.ds((g + 1) * GS, GS)],
                    dst_g.at[1 - gb], isem1).wait()
                pltpu.make_async_copy(rows_v.at[0], agg_sh.at[dst_g.at[gb, 0]],
                                      ssems[0]).wait()
                pltpu.async_copy(h_hbm.at[src_g.at[1 - gb, 0]], rows_v.at[0],
                                 sems[0])

    @pl.when(ng > 0)
    def _drain():
        pltpu.make_async_copy(rows_v.at[0], agg_sh.at[dst_g.at[0, 0]],
                              ssems[0]).wait()
        pltpu.make_async_copy(rows_v.at[1], agg_sh.at[dst_g.at[0, 0]],
                              ssems[1]).wait()
    plsc.subcore_barrier()

    pltpu.sync_copy(agg_sh.at[pl.ds(base, RPT)],
                    out_hbm.at[c, pl.ds(base, RPT)])


@functools.cache
def _sc_agg():
    return pl.kernel(
        _sc_agg_body,
        out_type=jax.ShapeDtypeStruct((NC, AGG_ROWS, F), jnp.float32),
        mesh=plsc.VectorSubcoreMesh(core_axis_name="c", subcore_axis_name="s",
                                    num_cores=NC, num_subcores=NS),
        scratch_types=[
            pltpu.VMEM((2, GS, CHUNK), jnp.int32),
            pltpu.VMEM((2, GS, CHUNK), jnp.int32),
            pltpu.VMEM((2, CHUNK, F), jnp.float32),
            pltpu.VMEM_SHARED((AGG_ROWS, F), jnp.float32),
            pltpu.SemaphoreType.DMA,
            pltpu.SemaphoreType.DMA,
            pltpu.SemaphoreType.DMA,
            pltpu.SemaphoreType.DMA,
            pltpu.SemaphoreType.DMA,
            pltpu.SemaphoreType.DMA,
        ],
    )


def _mlp_body(relu_last, h_ref, a0_ref, a1_ref, wa_ref, ba_ref, ga_ref,
              bea_ref, wb_ref, bb_ref, gb_ref, beb_ref, go_ref, beo_ref,
              out_ref):
    m = h_ref[...] + a0_ref[...] + a1_ref[...]
    t = jnp.dot(m, wa_ref[...], preferred_element_type=jnp.float32)
    t = (t + ba_ref[...]) * (ga_ref[...] * BN_INV) + bea_ref[...]
    t = jnp.maximum(t, 0.0)
    t = jnp.dot(t, wb_ref[...], preferred_element_type=jnp.float32)
    t = (t + bb_ref[...]) * (gb_ref[...] * BN_INV) + beb_ref[...]
    if relu_last:
        t = jnp.maximum(t, 0.0)
    t = t * (go_ref[...] * BN_INV) + beo_ref[...]
    if relu_last:
        t = jnp.maximum(t, 0.0)
    out_ref[...] = t


BLK = 1000


def _mlp(h, a0, a1, p, l, last):
    vec = lambda v: v.reshape(1, F)
    args = (h, a0, a1,
            p['w%da' % l], vec(p['b%da' % l]), vec(p['g%da' % l]),
            vec(p['be%da' % l]),
            p['w%db' % l], vec(p['b%db' % l]), vec(p['g%db' % l]),
            vec(p['be%db' % l]),
            vec(p['g%do' % l]), vec(p['be%do' % l]))
    row_spec = pl.BlockSpec((BLK, F), lambda i: (i, 0))
    full = lambda a: pl.BlockSpec(a.shape, lambda i: (0, 0))
    return pl.pallas_call(
        functools.partial(_mlp_body, not last),
        grid=(N // BLK,),
        in_specs=[row_spec, row_spec, row_spec] + [full(a) for a in args[3:]],
        out_specs=row_spec,
        out_shape=jax.ShapeDtypeStruct((N, F), jnp.float32),
    )(*args)


def _layout_edges(idx, fill):
    pad = EPAD - E
    filler = fill + jnp.arange(pad, dtype=jnp.int32) % (AGG_ROWS - N)
    idx = jnp.concatenate([idx, filler])
    c0 = idx[:CAP0].reshape(NS, Q0, CHUNK)
    c1 = idx[CAP0:].reshape(NS, Q1, CHUNK)
    unused = fill + (jnp.arange(NC * NS * NGMAX * GS * CHUNK,
                                dtype=jnp.int32) % (AGG_ROWS - N)
                     ).reshape(NC, NS, NGMAX * GS, CHUNK)
    out = unused
    out = out.at[0, :, :Q0].set(c0)
    out = out.at[1, :, :Q1].set(c1)
    return out


def kernel(x, edge_index, params):
    src = _layout_edges(edge_index[0].astype(jnp.int32), 0)
    dst = _layout_edges(edge_index[1].astype(jnp.int32), N)
    h = x
    for l in range(NUM_LAYERS):
        aggs = _sc_agg()(h, src, dst)
        h = _mlp(h, aggs[0], aggs[1], params, l, last=(l == NUM_LAYERS - 1))
    return h


# speedup vs baseline: 3.9921x; 1.0061x over previous
import functools

import jax
import jax.numpy as jnp
from jax import lax
from jax.experimental import pallas as pl
from jax.experimental.pallas import tpu as pltpu
from jax.experimental.pallas import tpu_sc as plsc

N = 10000
E = 320000
F = 128
NUM_LAYERS = 3
BN_EPS = 1e-5
BN_INV = 1.0 / (1.0 + BN_EPS) ** 0.5

NC = 2
NS = 16
CHUNK = 128
GS = 16
NG0 = 5
NG1 = 5
NGMAX = max(NG0, NG1)
Q0, Q1 = NG0 * GS, NG1 * GS
CAP0 = NS * Q0 * CHUNK
CAP1 = NS * Q1 * CHUNK
EPAD = CAP0 + CAP1
AGG_ROWS = 10240
RPT = AGG_ROWS // NS
ZCOPIES = RPT // CHUNK


def _sc_agg_body(h_hbm, src_hbm, dst_hbm, out_hbm, src_g, dst_g, rows_v,
                 agg_sh, sem0, sem1, isem0, isem1, ssem0, ssem1):
    c = lax.axis_index("c")
    s = lax.axis_index("s")
    sems = (sem0, sem1)
    ssems = (ssem0, ssem1)
    ng = jnp.where(c == 0, NG0, NG1)

    pltpu.sync_copy(src_hbm.at[c, s, pl.ds(0, GS)], src_g.at[0])
    pltpu.sync_copy(dst_hbm.at[c, s, pl.ds(0, GS)], dst_g.at[0])

    def zbody(i, _):
        for j in range(8):
            rows_v[0, i, pl.ds(j * 16, 16)] = jnp.zeros((16,), jnp.float32)
        return 0
    lax.fori_loop(0, CHUNK, zbody, 0)
    base = s * RPT
    for k in range(ZCOPIES):
        pltpu.sync_copy(rows_v.at[0], agg_sh.at[pl.ds(base + k * CHUNK,
                                                      CHUNK)])
    pltpu.async_copy(h_hbm.at[src_g.at[0, 0]], rows_v.at[0], sems[0])
    plsc.subcore_barrier()

    for g in range(NGMAX):
        gb = g % 2

        if g + 1 < NGMAX:
            @pl.when(g + 1 < ng)
            def _load_next(gb=gb, g=g):
                pltpu.async_copy(src_hbm.at[c, s, pl.ds((g + 1) * GS, GS)],
                                 src_g.at[1 - gb], isem0)
                pltpu.async_copy(dst_hbm.at[c, s, pl.ds((g + 1) * GS, GS)],
                                 dst_g.at[1 - gb], isem1)

        @pl.when(g < ng)
        def _run_group(gb=gb, g=g):
            first_group = g == 0

            def pair_body(i, _):
                for b in range(2):
                    k = 2 * i + b

                    @pl.when(k + 1 < GS)
                    def _start():
                        def _wait_scatter():
                            pltpu.make_async_copy(
                                rows_v.at[1 - b],
                                agg_sh.at[dst_g.at[gb, k]],
                                ssems[1 - b]).wait()
                        if first_group:
                            pl.when(k >= 1)(_wait_scatter)
                        else:
                            _wait_scatter()
                        pltpu.async_copy(h_hbm.at[src_g.at[gb, k + 1]],
                                         rows_v.at[1 - b], sems[1 - b])

                    pltpu.make_async_copy(h_hbm.at[src_g.at[gb, k]],
                                          rows_v.at[b], sems[b]).wait()
                    pltpu.async_copy(rows_v.at[b], agg_sh.at[dst_g.at[gb, k]],
                                     ssems[b], add=True)
                return 0
            lax.fori_loop(0, GS // 2, pair_body, 0)

        if g + 1 < NGMAX:
            @pl.when(g + 1 < ng)
            def _next_group_prime(gb=gb, g=g):
                pltpu.make_async_copy(
                    src_hbm.at[c, s, pl.ds((g + 1) * GS, GS)],
                    src_g.at[1 - gb], isem0).wait()
                pltpu.make_async_copy(
                    dst_hbm.at[c, s, pl.ds((g + 1) * GS, GS)],
                    dst_g.at[1 - gb], isem1).wait()
                pltpu.make_async_copy(rows_v.at[0], agg_sh.at[dst_g.at[gb, 0]],
                                      ssems[0]).wait()
                pltpu.async_copy(h_hbm.at[src_g.at[1 - gb, 0]], rows_v.at[0],
                                 sems[0])

    @pl.when(ng > 0)
    def _drain():
        pltpu.make_async_copy(rows_v.at[0], agg_sh.at[dst_g.at[0, 0]],
                              ssems[0]).wait()
        pltpu.make_async_copy(rows_v.at[1], agg_sh.at[dst_g.at[0, 0]],
                              ssems[1]).wait()
    plsc.subcore_barrier()

    pltpu.sync_copy(agg_sh.at[pl.ds(base, RPT)],
                    out_hbm.at[c, pl.ds(base, RPT)])


@functools.cache
def _sc_agg():
    return pl.kernel(
        _sc_agg_body,
        out_type=jax.ShapeDtypeStruct((NC, AGG_ROWS, F), jnp.float32),
        mesh=plsc.VectorSubcoreMesh(core_axis_name="c", subcore_axis_name="s",
                                    num_cores=NC, num_subcores=NS),
        scratch_types=[
            pltpu.VMEM((2, GS, CHUNK), jnp.int32),
            pltpu.VMEM((2, GS, CHUNK), jnp.int32),
            pltpu.VMEM((2, CHUNK, F), jnp.float32),
            pltpu.VMEM_SHARED((AGG_ROWS, F), jnp.float32),
            pltpu.SemaphoreType.DMA,
            pltpu.SemaphoreType.DMA,
            pltpu.SemaphoreType.DMA,
            pltpu.SemaphoreType.DMA,
            pltpu.SemaphoreType.DMA,
            pltpu.SemaphoreType.DMA,
        ],
    )


def _mlp_body(relu_last, h_ref, a0_ref, a1_ref, wa_ref, ba_ref, ga_ref,
              bea_ref, wb_ref, bb_ref, gb_ref, beb_ref, go_ref, beo_ref,
              out_ref):
    m = h_ref[...] + a0_ref[...] + a1_ref[...]
    t = jnp.dot(m, wa_ref[...], preferred_element_type=jnp.float32)
    t = (t + ba_ref[...]) * (ga_ref[...] * BN_INV) + bea_ref[...]
    t = jnp.maximum(t, 0.0)
    t = jnp.dot(t, wb_ref[...], preferred_element_type=jnp.float32)
    t = (t + bb_ref[...]) * (gb_ref[...] * BN_INV) + beb_ref[...]
    if relu_last:
        t = jnp.maximum(t, 0.0)
    t = t * (go_ref[...] * BN_INV) + beo_ref[...]
    if relu_last:
        t = jnp.maximum(t, 0.0)
    out_ref[...] = t


BLK = 1000


def _mlp(h, a0, a1, p, l, last):
    vec = lambda v: v.reshape(1, F)
    args = (h, a0, a1,
            p['w%da' % l], vec(p['b%da' % l]), vec(p['g%da' % l]),
            vec(p['be%da' % l]),
            p['w%db' % l], vec(p['b%db' % l]), vec(p['g%db' % l]),
            vec(p['be%db' % l]),
            vec(p['g%do' % l]), vec(p['be%do' % l]))
    row_spec = pl.BlockSpec((BLK, F), lambda i: (i, 0))
    full = lambda a: pl.BlockSpec(a.shape, lambda i: (0, 0))
    return pl.pallas_call(
        functools.partial(_mlp_body, not last),
        grid=(N // BLK,),
        in_specs=[row_spec, row_spec, row_spec] + [full(a) for a in args[3:]],
        out_specs=row_spec,
        out_shape=jax.ShapeDtypeStruct((N, F), jnp.float32),
    )(*args)


def _layout_edges(idx, fill):
    pad = EPAD - E
    filler = fill + jnp.arange(pad, dtype=jnp.int32) % (AGG_ROWS - N)
    idx = jnp.concatenate([idx, filler])
    assert Q0 == Q1
    return idx.reshape(NC, NS, Q0, CHUNK)


def kernel(x, edge_index, params):
    src = _layout_edges(edge_index[0].astype(jnp.int32), 0)
    dst = _layout_edges(edge_index[1].astype(jnp.int32), N)
    h = x
    for l in range(NUM_LAYERS):
        aggs = _sc_agg()(h, src, dst)
        h = _mlp(h, aggs[0], aggs[1], params, l, last=(l == NUM_LAYERS - 1))
    return h


# speedup vs baseline: 4.2233x; 1.0579x over previous
import functools

import jax
import jax.numpy as jnp
from jax import lax
from jax.experimental import pallas as pl
from jax.experimental.pallas import tpu as pltpu
from jax.experimental.pallas import tpu_sc as plsc

N = 10000
E = 320000
F = 128
NUM_LAYERS = 3
BN_EPS = 1e-5
BN_INV = 1.0 / (1.0 + BN_EPS) ** 0.5

NC = 2
NS = 16
CHUNK = 128
GS = 16
NG0 = 5
NG1 = 5
NGMAX = max(NG0, NG1)
Q0, Q1 = NG0 * GS, NG1 * GS
CAP0 = NS * Q0 * CHUNK
CAP1 = NS * Q1 * CHUNK
EPAD = CAP0 + CAP1
AGG_ROWS = 10240
RPT = AGG_ROWS // NS
ZCOPIES = RPT // CHUNK


def _sc_agg_body(h_hbm, src_hbm, dst_hbm, out_hbm, src_g, dst_g, rows_v,
                 agg_sh, sem0, sem1, isem0, isem1, ssem0, ssem1):
    c = lax.axis_index("c")
    s = lax.axis_index("s")
    sems = (sem0, sem1)
    ssems = (ssem0, ssem1)
    ng = jnp.where(c == 0, NG0, NG1)

    pltpu.sync_copy(src_hbm.at[c, s, pl.ds(0, GS)], src_g.at[0])
    pltpu.sync_copy(dst_hbm.at[c, s, pl.ds(0, GS)], dst_g.at[0])

    def zbody(i, _):
        for j in range(8):
            rows_v[0, i, pl.ds(j * 16, 16)] = jnp.zeros((16,), jnp.float32)
        return 0
    lax.fori_loop(0, CHUNK, zbody, 0)
    base = s * RPT
    for k in range(ZCOPIES):
        pltpu.sync_copy(rows_v.at[0], agg_sh.at[pl.ds(base + k * CHUNK,
                                                      CHUNK)])
    pltpu.async_copy(h_hbm.at[src_g.at[0, 0]], rows_v.at[0], sems[0])
    plsc.subcore_barrier()

    for g in range(NGMAX):
        gb = g % 2

        if g + 1 < NGMAX:
            @pl.when(g + 1 < ng)
            def _load_next(gb=gb, g=g):
                pltpu.async_copy(src_hbm.at[c, s, pl.ds((g + 1) * GS, GS)],
                                 src_g.at[1 - gb], isem0)
                pltpu.async_copy(dst_hbm.at[c, s, pl.ds((g + 1) * GS, GS)],
                                 dst_g.at[1 - gb], isem1)

        @pl.when(g < ng)
        def _run_group(gb=gb, g=g):
            first_group = g == 0

            def pair_body(i, _):
                for b in range(2):
                    k = 2 * i + b

                    @pl.when(k + 1 < GS)
                    def _start():
                        def _wait_scatter():
                            pltpu.make_async_copy(
                                rows_v.at[1 - b],
                                agg_sh.at[dst_g.at[gb, k]],
                                ssems[1 - b]).wait()
                        if first_group:
                            pl.when(k >= 1)(_wait_scatter)
                        else:
                            _wait_scatter()
                        pltpu.async_copy(h_hbm.at[src_g.at[gb, k + 1]],
                                         rows_v.at[1 - b], sems[1 - b])

                    pltpu.make_async_copy(h_hbm.at[src_g.at[gb, k]],
                                          rows_v.at[b], sems[b]).wait()
                    pltpu.async_copy(rows_v.at[b], agg_sh.at[dst_g.at[gb, k]],
                                     ssems[b], add=True)
                return 0
            lax.fori_loop(0, GS // 2, pair_body, 0)

        if g + 1 < NGMAX:
            @pl.when(g + 1 < ng)
            def _next_group_prime(gb=gb, g=g):
                pltpu.make_async_copy(
                    src_hbm.at[c, s, pl.ds((g + 1) * GS, GS)],
                    src_g.at[1 - gb], isem0).wait()
                pltpu.make_async_copy(
                    dst_hbm.at[c, s, pl.ds((g + 1) * GS, GS)],
                    dst_g.at[1 - gb], isem1).wait()
                pltpu.make_async_copy(rows_v.at[0], agg_sh.at[dst_g.at[gb, 0]],
                                      ssems[0]).wait()
                pltpu.async_copy(h_hbm.at[src_g.at[1 - gb, 0]], rows_v.at[0],
                                 sems[0])

    @pl.when(ng > 0)
    def _drain():
        pltpu.make_async_copy(rows_v.at[0], agg_sh.at[dst_g.at[0, 0]],
                              ssems[0]).wait()
        pltpu.make_async_copy(rows_v.at[1], agg_sh.at[dst_g.at[0, 0]],
                              ssems[1]).wait()
    plsc.subcore_barrier()

    pltpu.sync_copy(agg_sh.at[pl.ds(base, RPT)],
                    out_hbm.at[c, pl.ds(base, RPT)])


@functools.cache
def _sc_agg():
    return pl.kernel(
        _sc_agg_body,
        out_type=jax.ShapeDtypeStruct((NC, AGG_ROWS, F), jnp.float32),
        mesh=plsc.VectorSubcoreMesh(core_axis_name="c", subcore_axis_name="s",
                                    num_cores=NC, num_subcores=NS),
        scratch_types=[
            pltpu.VMEM((2, GS, CHUNK), jnp.int32),
            pltpu.VMEM((2, GS, CHUNK), jnp.int32),
            pltpu.VMEM((2, CHUNK, F), jnp.float32),
            pltpu.VMEM_SHARED((AGG_ROWS, F), jnp.float32),
            pltpu.SemaphoreType.DMA,
            pltpu.SemaphoreType.DMA,
            pltpu.SemaphoreType.DMA,
            pltpu.SemaphoreType.DMA,
            pltpu.SemaphoreType.DMA,
            pltpu.SemaphoreType.DMA,
        ],
    )


def _mlp_body(relu_last, h_ref, a0_ref, a1_ref, wa_ref, ba_ref, ga_ref,
              bea_ref, wb_ref, bb_ref, gb_ref, beb_ref, go_ref, beo_ref,
              out_ref):
    m = h_ref[...] + a0_ref[0] + a1_ref[0]
    t = jnp.dot(m, wa_ref[...], preferred_element_type=jnp.float32)
    t = (t + ba_ref[...]) * (ga_ref[...] * BN_INV) + bea_ref[...]
    t = jnp.maximum(t, 0.0)
    t = jnp.dot(t, wb_ref[...], preferred_element_type=jnp.float32)
    t = (t + bb_ref[...]) * (gb_ref[...] * BN_INV) + beb_ref[...]
    if relu_last:
        t = jnp.maximum(t, 0.0)
    t = t * (go_ref[...] * BN_INV) + beo_ref[...]
    if relu_last:
        t = jnp.maximum(t, 0.0)
    out_ref[...] = t


BLK = 1000


def _mlp(h, aggs, p, l, last):
    vec = lambda v: v.reshape(1, F)
    args = (h, aggs, aggs,
            p['w%da' % l], vec(p['b%da' % l]), vec(p['g%da' % l]),
            vec(p['be%da' % l]),
            p['w%db' % l], vec(p['b%db' % l]), vec(p['g%db' % l]),
            vec(p['be%db' % l]),
            vec(p['g%do' % l]), vec(p['be%do' % l]))
    row_spec = pl.BlockSpec((BLK, F), lambda i: (i, 0))
    agg0_spec = pl.BlockSpec((1, BLK, F), lambda i: (0, i, 0))
    agg1_spec = pl.BlockSpec((1, BLK, F), lambda i: (1, i, 0))
    full = lambda a: pl.BlockSpec(a.shape, lambda i: (0, 0))
    return pl.pallas_call(
        functools.partial(_mlp_body, not last),
        grid=(N // BLK,),
        in_specs=[row_spec, agg0_spec, agg1_spec]
        + [full(a) for a in args[3:]],
        out_specs=row_spec,
        out_shape=jax.ShapeDtypeStruct((N, F), jnp.float32),
    )(*args)


def _layout_edges(idx, fill):
    pad = EPAD - E
    filler = fill + jnp.arange(pad, dtype=jnp.int32) % (AGG_ROWS - N)
    idx = jnp.concatenate([idx, filler])
    assert Q0 == Q1
    return idx.reshape(NC, NS, Q0, CHUNK)


def kernel(x, edge_index, params):
    src = _layout_edges(edge_index[0].astype(jnp.int32), 0)
    dst = _layout_edges(edge_index[1].astype(jnp.int32), N)
    h = x
    for l in range(NUM_LAYERS):
        aggs = _sc_agg()(h, src, dst)
        h = _mlp(h, aggs, params, l, last=(l == NUM_LAYERS - 1))
    return h


# speedup vs baseline: 4.3043x; 1.0192x over previous
import functools

import jax
import jax.numpy as jnp
from jax import lax
from jax.experimental import pallas as pl
from jax.experimental.pallas import tpu as pltpu
from jax.experimental.pallas import tpu_sc as plsc

N = 10000
E = 320000
F = 128
NUM_LAYERS = 3
BN_EPS = 1e-5
BN_INV = 1.0 / (1.0 + BN_EPS) ** 0.5

NC = 2
NS = 16
CHUNK = 128
GS = 16
NG0 = 5
NG1 = 5
NGMAX = max(NG0, NG1)
Q0, Q1 = NG0 * GS, NG1 * GS
CAP0 = NS * Q0 * CHUNK
CAP1 = NS * Q1 * CHUNK
EPAD = CAP0 + CAP1
AGG_ROWS = 10240
RPT = AGG_ROWS // NS
ZCOPIES = RPT // CHUNK


def _sc_agg_body(h_hbm, src_hbm, dst_hbm, out_hbm, src_g, dst_g, rows_v,
                 agg_sh, sem0, sem1, isem0, isem1, ssem0, ssem1):
    c = lax.axis_index("c")
    s = lax.axis_index("s")
    sems = (sem0, sem1)
    ssems = (ssem0, ssem1)
    ng = jnp.where(c == 0, NG0, NG1)

    pltpu.sync_copy(src_hbm.at[c, s, pl.ds(0, GS)], src_g.at[0])
    pltpu.sync_copy(dst_hbm.at[c, s, pl.ds(0, GS)], dst_g.at[0])

    def zbody(i, _):
        for j in range(8):
            rows_v[0, i, pl.ds(j * 16, 16)] = jnp.zeros((16,), jnp.float32)
        return 0
    lax.fori_loop(0, CHUNK, zbody, 0)
    base = s * RPT
    for k in range(ZCOPIES):
        pltpu.sync_copy(rows_v.at[0], agg_sh.at[pl.ds(base + k * CHUNK,
                                                      CHUNK)])
    pltpu.async_copy(h_hbm.at[src_g.at[0, 0]], rows_v.at[0], sems[0])
    plsc.subcore_barrier()

    for g in range(NGMAX):
        gb = g % 2

        if g + 1 < NGMAX:
            @pl.when(g + 1 < ng)
            def _load_next(gb=gb, g=g):
                pltpu.async_copy(src_hbm.at[c, s, pl.ds((g + 1) * GS, GS)],
                                 src_g.at[1 - gb], isem0)
                pltpu.async_copy(dst_hbm.at[c, s, pl.ds((g + 1) * GS, GS)],
                                 dst_g.at[1 - gb], isem1)

        @pl.when(g < ng)
        def _run_group(gb=gb, g=g):
            first_group = g == 0

            def pair_body(i, _):
                for b in range(2):
                    k = 2 * i + b

                    @pl.when(k + 1 < GS)
                    def _start():
                        def _wait_scatter():
                            pltpu.make_async_copy(
                                rows_v.at[1 - b],
                                agg_sh.at[dst_g.at[gb, k]],
                                ssems[1 - b]).wait()
                        if first_group:
                            pl.when(k >= 1)(_wait_scatter)
                        else:
                            _wait_scatter()
                        pltpu.async_copy(h_hbm.at[src_g.at[gb, k + 1]],
                                         rows_v.at[1 - b], sems[1 - b])

                    pltpu.make_async_copy(h_hbm.at[src_g.at[gb, k]],
                                          rows_v.at[b], sems[b]).wait()
                    pltpu.async_copy(rows_v.at[b], agg_sh.at[dst_g.at[gb, k]],
                                     ssems[b], add=True)
                return 0
            lax.fori_loop(0, GS // 2, pair_body, 0)

        if g + 1 < NGMAX:
            @pl.when(g + 1 < ng)
            def _next_group_prime(gb=gb, g=g):
                pltpu.make_async_copy(
                    src_hbm.at[c, s, pl.ds((g + 1) * GS, GS)],
                    src_g.at[1 - gb], isem0).wait()
                pltpu.make_async_copy(
                    dst_hbm.at[c, s, pl.ds((g + 1) * GS, GS)],
                    dst_g.at[1 - gb], isem1).wait()
                pltpu.make_async_copy(rows_v.at[0], agg_sh.at[dst_g.at[gb, 0]],
                                      ssems[0]).wait()
                pltpu.async_copy(h_hbm.at[src_g.at[1 - gb, 0]], rows_v.at[0],
                                 sems[0])

    @pl.when(ng > 0)
    def _drain():
        pltpu.make_async_copy(rows_v.at[0], agg_sh.at[dst_g.at[0, 0]],
                              ssems[0]).wait()
        pltpu.make_async_copy(rows_v.at[1], agg_sh.at[dst_g.at[0, 0]],
                              ssems[1]).wait()
    plsc.subcore_barrier()

    pltpu.sync_copy(agg_sh.at[pl.ds(base, RPT)],
                    out_hbm.at[c, pl.ds(base, RPT)])


@functools.cache
def _sc_agg():
    return pl.kernel(
        _sc_agg_body,
        out_type=jax.ShapeDtypeStruct((NC, AGG_ROWS, F), jnp.float32),
        mesh=plsc.VectorSubcoreMesh(core_axis_name="c", subcore_axis_name="s",
                                    num_cores=NC, num_subcores=NS),
        scratch_types=[
            pltpu.VMEM((2, GS, CHUNK), jnp.int32),
            pltpu.VMEM((2, GS, CHUNK), jnp.int32),
            pltpu.VMEM((2, CHUNK, F), jnp.float32),
            pltpu.VMEM_SHARED((AGG_ROWS, F), jnp.float32),
            pltpu.SemaphoreType.DMA,
            pltpu.SemaphoreType.DMA,
            pltpu.SemaphoreType.DMA,
            pltpu.SemaphoreType.DMA,
            pltpu.SemaphoreType.DMA,
            pltpu.SemaphoreType.DMA,
        ],
    )


def _mlp_body(relu_last, h_ref, a0_ref, a1_ref, wa_ref, ba_ref, ga_ref,
              bea_ref, wb_ref, bb_ref, gb_ref, beb_ref, go_ref, beo_ref,
              out_ref):
    m = h_ref[...] + a0_ref[0] + a1_ref[0]
    t = jnp.dot(m, wa_ref[...], preferred_element_type=jnp.float32)
    t = (t + ba_ref[...]) * (ga_ref[...] * BN_INV) + bea_ref[...]
    t = jnp.maximum(t, 0.0)
    t = jnp.dot(t, wb_ref[...], preferred_element_type=jnp.float32)
    t = (t + bb_ref[...]) * (gb_ref[...] * BN_INV) + beb_ref[...]
    if relu_last:
        t = jnp.maximum(t, 0.0)
    t = t * (go_ref[...] * BN_INV) + beo_ref[...]
    if relu_last:
        t = jnp.maximum(t, 0.0)
    out_ref[...] = t


BLK = 2000


def _mlp(h, aggs, p, l, last):
    vec = lambda v: v.reshape(1, F)
    args = (h, aggs, aggs,
            p['w%da' % l], vec(p['b%da' % l]), vec(p['g%da' % l]),
            vec(p['be%da' % l]),
            p['w%db' % l], vec(p['b%db' % l]), vec(p['g%db' % l]),
            vec(p['be%db' % l]),
            vec(p['g%do' % l]), vec(p['be%do' % l]))
    row_spec = pl.BlockSpec((BLK, F), lambda i: (i, 0))
    agg0_spec = pl.BlockSpec((1, BLK, F), lambda i: (0, i, 0))
    agg1_spec = pl.BlockSpec((1, BLK, F), lambda i: (1, i, 0))
    full = lambda a: pl.BlockSpec(a.shape, lambda i: (0, 0))
    return pl.pallas_call(
        functools.partial(_mlp_body, not last),
        grid=(N // BLK,),
        in_specs=[row_spec, agg0_spec, agg1_spec]
        + [full(a) for a in args[3:]],
        out_specs=row_spec,
        out_shape=jax.ShapeDtypeStruct((N, F), jnp.float32),
    )(*args)


def _layout_edges(idx, fill):
    pad = EPAD - E
    filler = fill + jnp.arange(pad, dtype=jnp.int32) % (AGG_ROWS - N)
    idx = jnp.concatenate([idx, filler])
    assert Q0 == Q1
    return idx.reshape(NC, NS, Q0, CHUNK)


def kernel(x, edge_index, params):
    src = _layout_edges(edge_index[0].astype(jnp.int32), 0)
    dst = _layout_edges(edge_index[1].astype(jnp.int32), N)
    h = x
    for l in range(NUM_LAYERS):
        aggs = _sc_agg()(h, src, dst)
        h = _mlp(h, aggs, params, l, last=(l == NUM_LAYERS - 1))
    return h
